# baseline (speedup 1.0000x reference)
import functools

import jax
import jax.numpy as jnp
from jax import lax
from jax.experimental import pallas as pl
from jax.experimental.pallas import tpu as pltpu
from jax.experimental.pallas import tpu_sc as plsc

F = 26
V1 = 100001
D = 32
L = 16
NW = 32
CHUNK = 128
NBUF = 8
GDEPTH = 4


def _gather_kernel(per_w, idx_hbm, tab_hbm, out_hbm,
                   raw_v, idxflat, rows, *sems):
    gsems = sems[:NBUF]
    wsems = sems[NBUF:]
    n_chunks = per_w // CHUNK

    wid = lax.axis_index("c") * 16 + lax.axis_index("s")
    base = wid * per_w

    pltpu.sync_copy(idx_hbm.at[pl.ds(base, per_w)], raw_v)

    ovec0 = lax.iota(jnp.int32, L) * V1
    step = L * V1
    wrap = F * V1

    def compute_chunk(s, slot, ovec):
        for k in range(CHUNK // L):
            v = raw_v[pl.ds(s * CHUNK + k * L, L)]
            v = jnp.minimum(jnp.maximum(v, 0), V1 - 1)
            idxflat[slot, pl.ds(k * L, L)] = v + ovec
            ovec = ovec + step
            ovec = jnp.where(ovec >= wrap, ovec - wrap, ovec)
        return ovec

    def fire_gather(slot):
        pltpu.async_copy(tab_hbm.at[idxflat.at[slot]], rows.at[slot],
                         gsems[slot])

    def retire(s, slot):
        dst = out_hbm.at[pl.ds(base + s * CHUNK, CHUNK)]
        pltpu.make_async_copy(tab_hbm.at[idxflat.at[slot]], rows.at[slot],
                              gsems[slot]).wait()
        pltpu.async_copy(rows.at[slot], dst, wsems[slot])

    def wait_write(s, slot):
        dst = out_hbm.at[pl.ds(base + s * CHUNK, CHUNK)]
        pltpu.make_async_copy(rows.at[slot], dst, wsems[slot]).wait()

    ovec = ovec0
    for b in range(NBUF):
        ovec = compute_chunk(b, b, ovec)
        fire_gather(b)
        if b >= GDEPTH:
            retire(b - GDEPTH, b - GDEPTH)

    def body(g, ovec):
        for b in range(NBUF):
            s = g * NBUF + b
            wait_write(s - NBUF, b)
            ovec = compute_chunk(s, b, ovec)
            fire_gather(b)
            retire(s - GDEPTH, (b - GDEPTH) % NBUF)
        return ovec

    ovec = lax.fori_loop(1, n_chunks // NBUF, body, ovec)

    last = n_chunks
    for i in range(GDEPTH):
        s = last - GDEPTH + i
        retire(s, s % NBUF)
    for i in range(NBUF):
        s = last - NBUF + i
        wait_write(s, s % NBUF)


def kernel(int_feats, tables):
    B, nf = int_feats.shape
    assert nf == F and tables.shape == (F, V1, D)
    n_rows = B * F
    per_w = n_rows // NW
    assert n_rows % NW == 0 and per_w % F == 0 and per_w % CHUNK == 0
    assert (per_w // CHUNK) % NBUF == 0

    idx_flat = int_feats.reshape(n_rows).astype(jnp.int32)
    tab_flat = tables.reshape(F * V1, D)

    mesh = plsc.VectorSubcoreMesh(core_axis_name="c", subcore_axis_name="s",
                                  num_cores=2, num_subcores=16)
    run = pl.kernel(
        functools.partial(_gather_kernel, per_w),
        out_type=jax.ShapeDtypeStruct((n_rows, D), jnp.float32),
        mesh=mesh,
        scratch_types=(
            [pltpu.VMEM((per_w,), jnp.int32),
             pltpu.VMEM((NBUF, CHUNK), jnp.int32),
             pltpu.VMEM((NBUF, CHUNK, D), jnp.float32)]
            + [pltpu.SemaphoreType.DMA] * (2 * NBUF)
        ),
        compiler_params=pltpu.CompilerParams(use_tc_tiling_on_sc=False),
    )
    out = run(idx_flat, tab_flat)
    return out.reshape(B, F, D)


# speedup vs baseline: 49.0047x; 49.0047x over previous
import functools

import jax
import jax.numpy as jnp
from jax import lax
from jax.experimental import pallas as pl
from jax.experimental.pallas import tpu as pltpu
from jax.experimental.pallas import tpu_sc as plsc

F = 26
V1 = 100001
D = 32
L = 16
NW = 32
OCH = 2048


def _gather_kernel(B, idx_hbm, tab_hbm, out_hbm, idx_v, row_v, ob0, ob1,
                   *sems):
    obufs = (ob0, ob1)
    per_w = (F * D) // NW
    n_och = B // OCH
    wid = lax.axis_index("c") * 16 + lax.axis_index("s")
    p0 = wid * per_w

    def load_idx(f):
        pltpu.sync_copy(idx_hbm.at[f], idx_v)

        def clip16(i, _):
            for u in range(16):
                sl = pl.ds(i * 256 + u * L, L)
                idx_v[sl] = jnp.minimum(jnp.maximum(idx_v[sl], 0), V1 - 1)
            return 0

        lax.fori_loop(0, B // 256, clip16, 0)

    def task(t, _):
        p = p0 + t
        f = lax.shift_right_logical(p, 5)
        d = lax.bitwise_and(p, D - 1)

        @pl.when(jnp.logical_or(t == 0, d == 0))
        def _():
            load_idx(f)

        pltpu.sync_copy(tab_hbm.at[f, d], row_v)

        for c in range(n_och):
            buf = obufs[c % 2]
            prev = pltpu.make_async_copy(
                buf, out_hbm.at[0, 0, pl.ds(0, OCH)], sems[c % 2])
            if c >= 2:
                prev.wait()
            else:
                @pl.when(t > 0)
                def _():
                    prev.wait()

            def g8(i, _):
                for u in range(8):
                    j = i * 8 + u
                    iv = idx_v[pl.ds(c * OCH + j * L, L)]
                    buf[pl.ds(j * L, L)] = plsc.load_gather(row_v, [iv])
                return 0

            lax.fori_loop(0, OCH // (8 * L), g8, 0)
            pltpu.async_copy(
                buf, out_hbm.at[f, d, pl.ds(c * OCH, OCH)], sems[c % 2])
        return 0

    lax.fori_loop(0, per_w, task, 0)
    for k in range(2):
        pltpu.make_async_copy(
            obufs[k], out_hbm.at[0, 0, pl.ds(0, OCH)], sems[k]).wait()


def kernel(int_feats, tables):
    B, nf = int_feats.shape
    assert nf == F and tables.shape == (F, V1, D)
    assert (F * D) % NW == 0 and B % 256 == 0 and B % OCH == 0

    idx_t = int_feats.T
    tab_t = tables.transpose(0, 2, 1)

    mesh = plsc.VectorSubcoreMesh(core_axis_name="c", subcore_axis_name="s",
                                  num_cores=2, num_subcores=16)
    run = pl.kernel(
        functools.partial(_gather_kernel, B),
        out_type=jax.ShapeDtypeStruct((F, D, B), jnp.float32),
        mesh=mesh,
        scratch_types=(
            [pltpu.VMEM((B,), jnp.int32),
             pltpu.VMEM((V1,), jnp.float32),
             pltpu.VMEM((OCH,), jnp.float32),
             pltpu.VMEM((OCH,), jnp.float32)]
            + [pltpu.SemaphoreType.DMA] * 2
        ),
        compiler_params=pltpu.CompilerParams(needs_layout_passes=False),
    )
    out_t = run(idx_t, tab_t)
    return out_t.transpose(2, 0, 1)
